# Initial kernel scaffold; baseline (speedup 1.0000x reference)
import jax
import jax.numpy as jnp
from jax.experimental import pallas as pl


def kernel(x, edge_index, batch, bn_feat_g, bn_feat_b, W_feat, bnc_g0, bnc_b0, Wc0, bc0, bnc_g1, bnc_b1, Wc1, bc1, bnc_g2, bnc_b2, Wc2, bc2, bn_fc_g, bn_fc_b, W_fc, b_fc, bn_hid_g, bn_hid_b, W_cls, b_cls):
    raise NotImplementedError("write your pallas kernel here")


# speedup vs baseline: 6.4594x; 6.4594x over previous
import functools

import jax
import jax.numpy as jnp
from jax import lax
from jax.experimental import pallas as pl
from jax.experimental.pallas import tpu as pltpu
from jax.experimental.pallas import tpu_sc as plsc

_N = 10000
_E = 320000
_F = 128
_NG = 64
_NCLS = 10
_EPS = 1e-5

_NC = 2
_NS = 16
_NW = _NC * _NS
_CHUNK = 128
_EPW = 10240
_NCH = _EPW // _CHUNK
_EPAD = _NW * _EPW
_NPAD = 10112
_RPT = _NPAD // _NS


def _sc_mesh():
    return plsc.VectorSubcoreMesh(core_axis_name="c", subcore_axis_name="s")


def _deg_body(rdeg_hbm, ones_hbm, zero16_hbm, out_hbm, idx_v, ones_v, cnt_sh):
    cc = lax.axis_index("c")
    ss = lax.axis_index("s")
    wid = ss * _NC + cc
    pltpu.sync_copy(rdeg_hbm.at[wid], idx_v)
    pltpu.sync_copy(ones_hbm, ones_v)
    pltpu.sync_copy(zero16_hbm.at[pl.ds(ss * _RPT, _RPT)],
                    cnt_sh.at[pl.ds(ss * _RPT, _RPT)])
    plsc.subcore_barrier()

    def body(i, _):
        pltpu.sync_copy(ones_v, cnt_sh.at[idx_v.at[i]], add=True)
        return 0

    lax.fori_loop(0, _NCH, body, 0)
    plsc.subcore_barrier()
    pltpu.sync_copy(cnt_sh.at[pl.ds(ss * _RPT, _RPT)],
                    out_hbm.at[cc, pl.ds(ss * _RPT, _RPT)])


def _sc_degree(rdeg3, ones16, zeros16):
    k = pl.kernel(
        _deg_body,
        out_type=jax.ShapeDtypeStruct((_NC, _NPAD, 16), jnp.float32),
        mesh=_sc_mesh(),
        scratch_types=[
            pltpu.VMEM((_NCH, _CHUNK), jnp.int32),
            pltpu.VMEM((_CHUNK, 16), jnp.float32),
            pltpu.VMEM_SHARED((_NPAD, 16), jnp.float32),
        ],
    )
    return k(rdeg3, ones16, zeros16)


def _edge_body(h3_hbm, r_hbm, c_hbm, zeros_hbm, out_hbm,
               ridx_v, cidx_v, rows_v, agg_sh):
    cc = lax.axis_index("c")
    ss = lax.axis_index("s")
    wid = ss * _NC + cc
    pltpu.sync_copy(r_hbm.at[wid], ridx_v)
    pltpu.sync_copy(c_hbm.at[wid], cidx_v)
    pltpu.sync_copy(zeros_hbm.at[pl.ds(ss * _RPT, _RPT)],
                    agg_sh.at[pl.ds(ss * _RPT, _RPT)])
    plsc.subcore_barrier()

    def body(i, _):
        pltpu.sync_copy(h3_hbm.at[ridx_v.at[i]], rows_v)
        pltpu.sync_copy(rows_v, agg_sh.at[cidx_v.at[i]], add=True)
        return 0

    lax.fori_loop(0, _NCH, body, 0)
    plsc.subcore_barrier()
    pltpu.sync_copy(agg_sh.at[pl.ds(ss * _RPT, _RPT)],
                    out_hbm.at[cc, pl.ds(ss * _RPT, _RPT)])


def _sc_edge_pass(h3, r3, c3, zeros):
    k = pl.kernel(
        _edge_body,
        out_type=jax.ShapeDtypeStruct((_NC, _NPAD, _F), jnp.float32),
        mesh=_sc_mesh(),
        scratch_types=[
            pltpu.VMEM((_NCH, _CHUNK), jnp.int32),
            pltpu.VMEM((_NCH, _CHUNK), jnp.int32),
            pltpu.VMEM((_CHUNK, _F), jnp.float32),
            pltpu.VMEM_SHARED((_NPAD, _F), jnp.float32),
        ],
    )
    return k(h3, r3, c3, zeros)


def _bn(x, g, b):
    n = x.shape[0]
    m = jnp.sum(x, axis=0, keepdims=True) * (1.0 / n)
    d = x - m
    v = jnp.sum(d * d, axis=0, keepdims=True) * (1.0 / n)
    return g * d * lax.rsqrt(v + _EPS) + b


def _dis_col(cnt_ref):
    deg = cnt_ref[0, : _N, 0:1] + cnt_ref[1, : _N, 0:1] + 1.0
    return lax.rsqrt(deg)


def _pre_body(x_ref, cnt_ref, bfg_ref, bfb_ref, wf_ref, g0_ref, b0_ref,
              w0_ref, h3_ref):
    x = x_ref[...]
    h = _bn(x, bfg_ref[...], bfb_ref[...])
    h = jnp.maximum(jnp.dot(h, wf_ref[...],
                            preferred_element_type=jnp.float32), 0.0)
    t = _bn(h, g0_ref[...], b0_ref[...])
    h2 = jnp.dot(t, w0_ref[...], preferred_element_type=jnp.float32)
    h3_ref[...] = _dis_col(cnt_ref) * h2


def _tc_pre(x, cnt, bn_feat_g, bn_feat_b, W_feat, g0, b0, W0):
    return pl.pallas_call(
        _pre_body,
        out_shape=jax.ShapeDtypeStruct((_N, _F), jnp.float32),
    )(x, cnt, bn_feat_g, bn_feat_b, W_feat, g0, b0, W0)


def _mid_body(agg_ref, h3_ref, cnt_ref, bc_ref, g_ref, b_ref, w_ref, out_ref):
    dis = _dis_col(cnt_ref)
    agg = agg_ref[0, : _N, :] + agg_ref[1, : _N, :] + h3_ref[...]
    h = jnp.maximum(dis * agg + bc_ref[...], 0.0)
    t = _bn(h, g_ref[...], b_ref[...])
    h2 = jnp.dot(t, w_ref[...], preferred_element_type=jnp.float32)
    out_ref[...] = dis * h2


def _tc_mid(agg, h3, cnt, bc, g, b, W):
    return pl.pallas_call(
        _mid_body,
        out_shape=jax.ShapeDtypeStruct((_N, _F), jnp.float32),
    )(agg, h3, cnt, bc, g, b, W)


def _final_body(agg_ref, h3_ref, cnt_ref, bc_ref, oh_ref,
                gfc_ref, bfc_ref, wfc_ref, bfcb_ref,
                gh_ref, bh_ref, wcls_ref, bcls_ref, out_ref):
    dis = _dis_col(cnt_ref)
    agg = agg_ref[0, : _N, :] + agg_ref[1, : _N, :] + h3_ref[...]
    h = jnp.maximum(dis * agg + bc_ref[...], 0.0)
    pooled = jnp.dot(oh_ref[...], h, preferred_element_type=jnp.float32)
    t = _bn(pooled, gfc_ref[...], bfc_ref[...])
    t = jnp.maximum(jnp.dot(t, wfc_ref[...],
                            preferred_element_type=jnp.float32)
                    + bfcb_ref[...], 0.0)
    t = _bn(t, gh_ref[...], bh_ref[...])
    logits = jnp.dot(t, wcls_ref[...],
                     preferred_element_type=jnp.float32) + bcls_ref[...]
    mx = jnp.max(logits, axis=-1, keepdims=True)
    ex = jnp.exp(logits - mx)
    lse = mx + jnp.log(jnp.sum(ex, axis=-1, keepdims=True))
    out_ref[...] = logits - lse


def _tc_final(agg, h3, cnt, bc, oh, gfc, bfc, wfc, bfcb, gh, bh, wcls, bcls):
    return pl.pallas_call(
        _final_body,
        out_shape=jax.ShapeDtypeStruct((_NG, _NCLS), jnp.float32),
    )(agg, h3, cnt, bc, oh, gfc, bfc, wfc, bfcb, gh, bh, wcls, bcls)


def kernel(x, edge_index, batch, bn_feat_g, bn_feat_b, W_feat,
           bnc_g0, bnc_b0, Wc0, bc0, bnc_g1, bnc_b1, Wc1, bc1,
           bnc_g2, bnc_b2, Wc2, bc2, bn_fc_g, bn_fc_b, W_fc, b_fc,
           bn_hid_g, bn_hid_b, W_cls, b_cls):
    npad = _EPAD - _E
    r = edge_index[0]
    c = edge_index[1]
    r3 = jnp.concatenate([r, jnp.zeros((npad,), jnp.int32)]
                         ).reshape(_NW, _NCH, _CHUNK)
    c3 = jnp.concatenate([c, jnp.full((npad,), _N, jnp.int32)]
                         ).reshape(_NW, _NCH, _CHUNK)
    rdeg3 = jnp.concatenate([r, jnp.full((npad,), _N, jnp.int32)]
                            ).reshape(_NW, _NCH, _CHUNK)
    ones16 = jnp.ones((_CHUNK, 16), jnp.float32)
    zeros16 = jnp.zeros((_NPAD, 16), jnp.float32)
    zeros = jnp.zeros((_NPAD, _F), jnp.float32)
    oh = (jnp.arange(_NG, dtype=jnp.int32)[:, None] == batch[None, :]
          ).astype(jnp.float32)

    cnt = _sc_degree(rdeg3, ones16, zeros16)

    h3 = _tc_pre(x, cnt, bn_feat_g, bn_feat_b, W_feat, bnc_g0, bnc_b0, Wc0)
    agg = _sc_edge_pass(h3, r3, c3, zeros)
    h3 = _tc_mid(agg, h3, cnt, bc0, bnc_g1, bnc_b1, Wc1)
    agg = _sc_edge_pass(h3, r3, c3, zeros)
    h3 = _tc_mid(agg, h3, cnt, bc1, bnc_g2, bnc_b2, Wc2)
    agg = _sc_edge_pass(h3, r3, c3, zeros)
    return _tc_final(agg, h3, cnt, bc2, oh, bn_fc_g, bn_fc_b, W_fc, b_fc,
                     bn_hid_g, bn_hid_b, W_cls, b_cls)
